# baseline (speedup 1.0000x reference)
import functools

import jax
import jax.numpy as jnp
from jax import lax
from jax.experimental import pallas as pl
from jax.experimental.pallas import tpu as pltpu
from jax.experimental.pallas import tpu_sc as plsc

D = 768
MLPD = 3072
NE = 8
G = 2
S_DET = 2048
S_CLS = 256
C_DET = 512
C_CLS = 64
CT = C_DET + C_CLS
NVALID = G * NE * CT
NTRASH = 64
R_ROWS = NVALID + NTRASH
MBK = 1536
NMB = MLPD // MBK
NC = 2
NS = 16


def _router_body(x_ref, wr_ref, d0_ref, d1_ref, g0_ref, g1_ref, w0_ref,
                 w1_ref, *, S, C, slot_off):
    g = pl.program_id(0)
    x = x_ref[...]
    wr = wr_ref[...]
    logits = jnp.dot(x, wr, preferred_element_type=jnp.float32)
    m = jnp.max(logits, axis=1, keepdims=True)
    ex = jnp.exp(logits - m)
    gates = ex / jnp.sum(ex, axis=1, keepdims=True)
    idx8 = lax.broadcasted_iota(jnp.int32, (S, NE), 1)
    v0 = jnp.max(gates, axis=1, keepdims=True)
    i0 = jnp.min(jnp.where(gates >= v0, idx8, NE), axis=1, keepdims=True)
    oh0 = idx8 == i0
    masked = jnp.where(oh0, -1.0, gates)
    v1 = jnp.max(masked, axis=1, keepdims=True)
    i1 = jnp.min(jnp.where(masked >= v1, idx8, NE), axis=1, keepdims=True)
    oh1 = idx8 == i1
    oh0f = oh0.astype(jnp.float32)
    oh1f = oh1.astype(jnp.float32)

    SB = min(S, 256)
    r_io = lax.broadcasted_iota(jnp.int32, (SB, SB), 0)
    c_io = lax.broadcasted_iota(jnp.int32, (SB, SB), 1)
    tril = (r_io >= c_io).astype(jnp.float32)

    def block_cumsum(oh, carry):
        blocks = []
        for b in range(S // SB):
            blk = lax.slice_in_dim(oh, b * SB, (b + 1) * SB, axis=0)
            cum = jnp.dot(tril, blk, preferred_element_type=jnp.float32) + carry
            blocks.append(cum)
            carry = lax.slice_in_dim(cum, SB - 1, SB, axis=0)
        return (blocks[0] if len(blocks) == 1 else jnp.concatenate(blocks, 0),
                carry)

    cum0, tot0 = block_cumsum(oh0f, jnp.zeros((1, NE), jnp.float32))
    cum1, _ = block_cumsum(oh1f, tot0)

    pos0 = jnp.sum(cum0 * oh0f, axis=1, keepdims=True) - 1.0
    pos1 = jnp.sum(cum1 * oh1f, axis=1, keepdims=True) - 1.0
    keep0 = pos0 < C
    keep1 = pos1 < C
    p0 = jnp.clip(pos0, 0.0, C - 1.0).astype(jnp.int32)
    p1 = jnp.clip(pos1, 0.0, C - 1.0).astype(jnp.int32)
    base0 = (i0 * G + g) * CT + slot_off + p0
    base1 = (i1 * G + g) * CT + slot_off + p1
    siota = lax.broadcasted_iota(jnp.int32, (S, 1), 0)
    trash = NVALID + (siota % NTRASH)
    d0 = jnp.where(keep0, base0, trash)
    d1 = jnp.where(keep1, base1, trash)
    w0 = v0 * keep0.astype(jnp.float32)
    w1 = v1 * keep1.astype(jnp.float32)

    d0_ref[...] = d0.reshape(1, S, 1)
    d1_ref[...] = d1.reshape(1, S, 1)
    g0_ref[...] = base0.reshape(1, S, 1)
    g1_ref[...] = base1.reshape(1, S, 1)
    w0_ref[...] = jnp.broadcast_to(w0, (S, 16)).reshape(1, S, 16)
    w1_ref[...] = jnp.broadcast_to(w1, (S, 16)).reshape(1, S, 16)


def _router_call(x2, wr, S, C, slot_off):
    i_sds = jax.ShapeDtypeStruct((G, S, 1), jnp.int32)
    f_sds = jax.ShapeDtypeStruct((G, S, 16), jnp.float32)
    outs = pl.pallas_call(
        functools.partial(_router_body, S=S, C=C, slot_off=slot_off),
        grid=(G,),
        in_specs=[
            pl.BlockSpec((S, D), lambda g: (g, 0)),
            pl.BlockSpec((D, NE), lambda g: (0, 0)),
        ],
        out_specs=[pl.BlockSpec((1, S, 1), lambda g: (g, 0, 0))] * 4
        + [pl.BlockSpec((1, S, 16), lambda g: (g, 0, 0))] * 2,
        out_shape=[i_sds, i_sds, i_sds, i_sds, f_sds, f_sds],
    )(x2, wr)
    return ([o.reshape(-1) for o in outs[:4]]
            + [o.reshape(-1, 16) for o in outs[4:]])


def _dispatch_body(xd_hbm, xc_hbm, d0d_hbm, d1d_hbm, d0c_hbm,
                   d1c_hbm, xe_hbm, bv, iv, ivc):
    c = lax.axis_index("c")
    s = lax.axis_index("s")
    for ch in range(2):
        off = c * S_DET + s * 128 + ch * 64
        pltpu.sync_copy(xd_hbm.at[pl.ds(off, 64)], bv.at[pl.ds(0, 64)])
        pltpu.sync_copy(d0d_hbm.at[pl.ds(off, 64)], iv)
        pltpu.sync_copy(bv.at[pl.ds(0, 64)], xe_hbm.at[iv])
        pltpu.sync_copy(d1d_hbm.at[pl.ds(off, 64)], iv)
        pltpu.sync_copy(bv.at[pl.ds(0, 64)], xe_hbm.at[iv])
    offc = c * S_CLS + s * 16
    pltpu.sync_copy(xc_hbm.at[pl.ds(offc, 16)], bv.at[pl.ds(0, 16)])
    pltpu.sync_copy(d0c_hbm.at[pl.ds(offc, 16)], ivc)
    pltpu.sync_copy(bv.at[pl.ds(0, 16)], xe_hbm.at[ivc])
    pltpu.sync_copy(d1c_hbm.at[pl.ds(offc, 16)], ivc)
    pltpu.sync_copy(bv.at[pl.ds(0, 16)], xe_hbm.at[ivc])


def _dispatch_call(xd2, xc2, d0d, d1d, d0c, d1c):
    mesh = plsc.VectorSubcoreMesh(core_axis_name="c", subcore_axis_name="s")
    return pl.kernel(
        _dispatch_body,
        out_type=jax.ShapeDtypeStruct((R_ROWS, D), jnp.float32),
        mesh=mesh,
        scratch_types=[
            pltpu.VMEM((64, D), jnp.float32),
            pltpu.VMEM((64,), jnp.int32),
            pltpu.VMEM((16,), jnp.int32),
        ],
    )(xd2, xc2, d0d, d1d, d0c, d1c)


def _mlp_body(xe_ref, w1_ref, b1_ref, w2_ref, b2_ref, out_ref):
    mb = pl.program_id(1)

    @pl.when(mb == 0)
    def _():
        out_ref[...] = jnp.broadcast_to(b2_ref[0], (G * CT, D))

    h = jnp.dot(xe_ref[...].astype(jnp.bfloat16),
                w1_ref[0].astype(jnp.bfloat16),
                preferred_element_type=jnp.float32) + b1_ref[0]
    h = jax.nn.gelu(h).astype(jnp.bfloat16)
    out_ref[...] += jnp.dot(h, w2_ref[0].astype(jnp.bfloat16),
                            preferred_element_type=jnp.float32)


def _mlp_call(xe, w1, b1, w2, b2):
    return pl.pallas_call(
        _mlp_body,
        grid=(NE, NMB),
        in_specs=[
            pl.BlockSpec((G * CT, D), lambda e, mb: (e, 0)),
            pl.BlockSpec((1, D, MBK), lambda e, mb: (e, 0, mb)),
            pl.BlockSpec((1, 1, MBK), lambda e, mb: (e, 0, mb)),
            pl.BlockSpec((1, MBK, D), lambda e, mb: (e, mb, 0)),
            pl.BlockSpec((1, 1, D), lambda e, mb: (e, 0, 0)),
        ],
        out_specs=pl.BlockSpec((G * CT, D), lambda e, mb: (e, 0)),
        out_shape=jax.ShapeDtypeStruct((NVALID, D), jnp.float32),
        compiler_params=pltpu.CompilerParams(
            dimension_semantics=("parallel", "arbitrary")),
    )(xe, w1, b1.reshape(NE, 1, MLPD), w2, b2.reshape(NE, 1, D))


def _combine_body(ye_hbm, g0d_hbm, g1d_hbm, w0d_hbm, w1d_hbm, g0c_hbm,
                  g1c_hbm, w0c_hbm, w1c_hbm, outd_hbm, outc_hbm,
                  r0v, r1v, ov, iv0, iv1, wv0, wv1, gsem, wsem):
    c = lax.axis_index("c")
    s = lax.axis_index("s")
    w = s * NC + c

    def weighted_rows(n, r0, r1, roff, o, ooff, vw0, vw1, woff):
        def body(t4, _):
            t = t4 * 4
            for k in range(4):
                wb0 = vw0[woff + t + k, pl.ds(0, 16)]
                wb1 = vw1[woff + t + k, pl.ds(0, 16)]
                for j in range(D // 16):
                    a = r0[roff + t + k, pl.ds(j * 16, 16)]
                    b = r1[roff + t + k, pl.ds(j * 16, 16)]
                    o[ooff + t + k, pl.ds(j * 16, 16)] = a * wb0 + b * wb1
            return 0
        lax.fori_loop(0, n // 4, body, 0)

    toff = w * 128
    pltpu.sync_copy(g0d_hbm.at[pl.ds(toff, 128)], iv0)
    pltpu.sync_copy(g1d_hbm.at[pl.ds(toff, 128)], iv1)
    pltpu.sync_copy(w0d_hbm.at[pl.ds(toff, 128)], wv0)
    pltpu.sync_copy(w1d_hbm.at[pl.ds(toff, 128)], wv1)

    CH = 16
    def gath(ch, slot):
        return (pltpu.make_async_copy(
                    ye_hbm.at[iv0.at[pl.ds(ch * CH, CH)]],
                    r0v.at[pl.ds(slot * CH, CH)], gsem),
                pltpu.make_async_copy(
                    ye_hbm.at[iv1.at[pl.ds(ch * CH, CH)]],
                    r1v.at[pl.ds(slot * CH, CH)], gsem))

    def wrt(ch, slot):
        return pltpu.make_async_copy(
            ov.at[pl.ds(slot * CH, CH)],
            outd_hbm.at[pl.ds(toff + ch * CH, CH)], wsem)

    for c0, c1 in (gath(0, 0), gath(1, 1)):
        c0.start(); c1.start()
    for ch in range(8):
        gs = ch % 2
        os = ch % 2
        c0, c1 = gath(ch, gs)
        c0.wait(); c1.wait()
        if ch >= 2:
            wrt(ch - 2, os).wait()
        weighted_rows(CH, r0v, r1v, gs * CH, ov, os * CH, wv0, wv1, ch * CH)
        wrt(ch, os).start()
        if ch < 6:
            n0, n1 = gath(ch + 2, gs)
            n0.start(); n1.start()

    toff = w * 16
    pltpu.sync_copy(g0c_hbm.at[pl.ds(toff, 16)],
                    iv0.at[pl.ds(0, 16)])
    pltpu.sync_copy(g1c_hbm.at[pl.ds(toff, 16)], iv1.at[pl.ds(0, 16)])
    pltpu.sync_copy(w0c_hbm.at[pl.ds(toff, 16)], wv0.at[pl.ds(0, 16)])
    pltpu.sync_copy(w1c_hbm.at[pl.ds(toff, 16)], wv1.at[pl.ds(0, 16)])
    pltpu.sync_copy(ye_hbm.at[iv0.at[pl.ds(0, 16)]], r0v.at[pl.ds(0, 16)])
    pltpu.sync_copy(ye_hbm.at[iv1.at[pl.ds(0, 16)]], r1v.at[pl.ds(0, 16)])
    wrt(6, 0).wait()
    wrt(7, 1).wait()
    weighted_rows(16, r0v, r1v, 0, r0v, 16, wv0, wv1, 0)
    pltpu.sync_copy(r0v.at[pl.ds(16, 16)], outc_hbm.at[pl.ds(toff, 16)])


def _combine_call(ye, g0d, g1d, w0d, w1d, g0c, g1c, w0c, w1c):
    mesh = plsc.VectorSubcoreMesh(core_axis_name="c", subcore_axis_name="s")
    return pl.kernel(
        _combine_body,
        out_type=(jax.ShapeDtypeStruct((G * S_DET, D), jnp.float32),
                  jax.ShapeDtypeStruct((G * S_CLS, D), jnp.float32)),
        mesh=mesh,
        scratch_types=[
            pltpu.VMEM((32, D), jnp.float32),
            pltpu.VMEM((32, D), jnp.float32),
            pltpu.VMEM((32, D), jnp.float32),
            pltpu.VMEM((128,), jnp.int32),
            pltpu.VMEM((128,), jnp.int32),
            pltpu.VMEM((128, 16), jnp.float32),
            pltpu.VMEM((128, 16), jnp.float32),
            pltpu.SemaphoreType.DMA,
            pltpu.SemaphoreType.DMA,
        ],
    )(ye, g0d, g1d, w0d, w1d, g0c, g1c, w0c, w1c)


def kernel(inputs_det, inputs_cls, w_router_det, w_router_cls, w1, b1, w2, b2):
    xd2 = inputs_det.reshape(G * S_DET, D)
    xc2 = inputs_cls.reshape(G * S_CLS, D)
    d0d, d1d, g0d, g1d, w0d, w1d = _router_call(xd2, w_router_det,
                                                S_DET, C_DET, 0)
    d0c, d1c, g0c, g1c, w0c, w1c = _router_call(xc2, w_router_cls,
                                                S_CLS, C_CLS, C_DET)
    xe = _dispatch_call(xd2, xc2, d0d, d1d, d0c, d1c)
    ye = _mlp_call(xe, w1, b1, w2, b2)
    outd2, outc2 = _combine_call(ye, g0d, g1d, w0d, w1d, g0c, g1c, w0c, w1c)
    return outd2.reshape(inputs_det.shape), outc2.reshape(inputs_cls.shape)


# speedup vs baseline: 1.0421x; 1.0421x over previous
import functools

import jax
import jax.numpy as jnp
from jax import lax
from jax.experimental import pallas as pl
from jax.experimental.pallas import tpu as pltpu
from jax.experimental.pallas import tpu_sc as plsc

D = 768
MLPD = 3072
NE = 8
G = 2
S_DET = 2048
S_CLS = 256
C_DET = 512
C_CLS = 64
CT = C_DET + C_CLS
NVALID = G * NE * CT
NTRASH = 64
R_ROWS = NVALID + NTRASH
MBK = 1536
NMB = MLPD // MBK
NC = 2
NS = 16


def _router_body(x_ref, wr_ref, d0_ref, d1_ref, g0_ref, g1_ref, w0_ref,
                 w1_ref, *, S, C, slot_off):
    g = pl.program_id(0)
    x = x_ref[...]
    wr = wr_ref[...]
    logits = jnp.dot(x, wr, preferred_element_type=jnp.float32)
    m = jnp.max(logits, axis=1, keepdims=True)
    ex = jnp.exp(logits - m)
    gates = ex / jnp.sum(ex, axis=1, keepdims=True)
    idx8 = lax.broadcasted_iota(jnp.int32, (S, NE), 1)
    v0 = jnp.max(gates, axis=1, keepdims=True)
    i0 = jnp.min(jnp.where(gates >= v0, idx8, NE), axis=1, keepdims=True)
    oh0 = idx8 == i0
    masked = jnp.where(oh0, -1.0, gates)
    v1 = jnp.max(masked, axis=1, keepdims=True)
    i1 = jnp.min(jnp.where(masked >= v1, idx8, NE), axis=1, keepdims=True)
    oh1 = idx8 == i1
    oh0f = oh0.astype(jnp.float32)
    oh1f = oh1.astype(jnp.float32)

    SB = min(S, 256)
    r_io = lax.broadcasted_iota(jnp.int32, (SB, SB), 0)
    c_io = lax.broadcasted_iota(jnp.int32, (SB, SB), 1)
    tril = (r_io >= c_io).astype(jnp.float32)

    def block_cumsum(oh, carry):
        blocks = []
        for b in range(S // SB):
            blk = lax.slice_in_dim(oh, b * SB, (b + 1) * SB, axis=0)
            cum = jnp.dot(tril, blk, preferred_element_type=jnp.float32) + carry
            blocks.append(cum)
            carry = lax.slice_in_dim(cum, SB - 1, SB, axis=0)
        return (blocks[0] if len(blocks) == 1 else jnp.concatenate(blocks, 0),
                carry)

    cum0, tot0 = block_cumsum(oh0f, jnp.zeros((1, NE), jnp.float32))
    cum1, _ = block_cumsum(oh1f, tot0)

    pos0 = jnp.sum(cum0 * oh0f, axis=1, keepdims=True) - 1.0
    pos1 = jnp.sum(cum1 * oh1f, axis=1, keepdims=True) - 1.0
    keep0 = pos0 < C
    keep1 = pos1 < C
    p0 = jnp.clip(pos0, 0.0, C - 1.0).astype(jnp.int32)
    p1 = jnp.clip(pos1, 0.0, C - 1.0).astype(jnp.int32)
    base0 = (i0 * G + g) * CT + slot_off + p0
    base1 = (i1 * G + g) * CT + slot_off + p1
    siota = lax.broadcasted_iota(jnp.int32, (S, 1), 0)
    trash = NVALID + (siota % NTRASH)
    d0 = jnp.where(keep0, base0, trash)
    d1 = jnp.where(keep1, base1, trash)
    w0 = v0 * keep0.astype(jnp.float32)
    w1 = v1 * keep1.astype(jnp.float32)

    d0_ref[...] = d0.reshape(1, S, 1)
    d1_ref[...] = d1.reshape(1, S, 1)
    g0_ref[...] = base0.reshape(1, S, 1)
    g1_ref[...] = base1.reshape(1, S, 1)
    w0_ref[...] = jnp.broadcast_to(w0, (S, 16)).reshape(1, S, 16)
    w1_ref[...] = jnp.broadcast_to(w1, (S, 16)).reshape(1, S, 16)


def _router_call(x2, wr, S, C, slot_off):
    i_sds = jax.ShapeDtypeStruct((G, S, 1), jnp.int32)
    f_sds = jax.ShapeDtypeStruct((G, S, 16), jnp.float32)
    outs = pl.pallas_call(
        functools.partial(_router_body, S=S, C=C, slot_off=slot_off),
        grid=(G,),
        in_specs=[
            pl.BlockSpec((S, D), lambda g: (g, 0)),
            pl.BlockSpec((D, NE), lambda g: (0, 0)),
        ],
        out_specs=[pl.BlockSpec((1, S, 1), lambda g: (g, 0, 0))] * 4
        + [pl.BlockSpec((1, S, 16), lambda g: (g, 0, 0))] * 2,
        out_shape=[i_sds, i_sds, i_sds, i_sds, f_sds, f_sds],
    )(x2, wr)
    return ([o.reshape(-1) for o in outs[:4]]
            + [o.reshape(-1, 16) for o in outs[4:]])


def _dispatch_body(xd_hbm, xc_hbm, d0d_hbm, d1d_hbm, d0c_hbm,
                   d1c_hbm, xe_hbm, bv0, bv1, bc,
                   iv00, iv01, iv10, iv11, ivc0, ivc1, rsem, ssem):
    c = lax.axis_index("c")
    s = lax.axis_index("s")
    off = c * S_DET + s * 128
    offc = c * S_CLS + s * 16
    r0 = pltpu.make_async_copy(xd_hbm.at[pl.ds(off, 64)], bv0, rsem)
    r1 = pltpu.make_async_copy(xd_hbm.at[pl.ds(off + 64, 64)], bv1, rsem)
    rc = pltpu.make_async_copy(xc_hbm.at[pl.ds(offc, 16)], bc, rsem)
    r0.start(); r1.start(); rc.start()
    pltpu.sync_copy(d0d_hbm.at[pl.ds(off, 64)], iv00)
    pltpu.sync_copy(d1d_hbm.at[pl.ds(off, 64)], iv01)
    pltpu.sync_copy(d0d_hbm.at[pl.ds(off + 64, 64)], iv10)
    pltpu.sync_copy(d1d_hbm.at[pl.ds(off + 64, 64)], iv11)
    pltpu.sync_copy(d0c_hbm.at[pl.ds(offc, 16)], ivc0)
    pltpu.sync_copy(d1c_hbm.at[pl.ds(offc, 16)], ivc1)
    scs = [pltpu.make_async_copy(bv0, xe_hbm.at[iv00], ssem),
           pltpu.make_async_copy(bv0, xe_hbm.at[iv01], ssem),
           pltpu.make_async_copy(bv1, xe_hbm.at[iv10], ssem),
           pltpu.make_async_copy(bv1, xe_hbm.at[iv11], ssem),
           pltpu.make_async_copy(bc, xe_hbm.at[ivc0], ssem),
           pltpu.make_async_copy(bc, xe_hbm.at[ivc1], ssem)]
    r0.wait()
    scs[0].start(); scs[1].start()
    r1.wait()
    scs[2].start(); scs[3].start()
    rc.wait()
    scs[4].start(); scs[5].start()
    for sc in scs:
        sc.wait()


def _dispatch_call(xd2, xc2, d0d, d1d, d0c, d1c):
    mesh = plsc.VectorSubcoreMesh(core_axis_name="c", subcore_axis_name="s")
    return pl.kernel(
        _dispatch_body,
        out_type=jax.ShapeDtypeStruct((R_ROWS, D), jnp.float32),
        mesh=mesh,
        scratch_types=[
            pltpu.VMEM((64, D), jnp.float32),
            pltpu.VMEM((64, D), jnp.float32),
            pltpu.VMEM((16, D), jnp.float32),
            pltpu.VMEM((64,), jnp.int32),
            pltpu.VMEM((64,), jnp.int32),
            pltpu.VMEM((64,), jnp.int32),
            pltpu.VMEM((64,), jnp.int32),
            pltpu.VMEM((16,), jnp.int32),
            pltpu.VMEM((16,), jnp.int32),
            pltpu.SemaphoreType.DMA,
            pltpu.SemaphoreType.DMA,
        ],
    )(xd2, xc2, d0d, d1d, d0c, d1c)


def _mlp_body(xe_ref, w1_ref, b1_ref, w2_ref, b2_ref, out_ref):
    mb = pl.program_id(1)

    @pl.when(mb == 0)
    def _():
        out_ref[...] = jnp.broadcast_to(b2_ref[0], (G * CT, D))

    h = jnp.dot(xe_ref[...].astype(jnp.bfloat16),
                w1_ref[0].astype(jnp.bfloat16),
                preferred_element_type=jnp.float32) + b1_ref[0]
    h = jax.nn.gelu(h).astype(jnp.bfloat16)
    out_ref[...] += jnp.dot(h, w2_ref[0].astype(jnp.bfloat16),
                            preferred_element_type=jnp.float32)


def _mlp_call(xe, w1, b1, w2, b2):
    return pl.pallas_call(
        _mlp_body,
        grid=(NE, NMB),
        in_specs=[
            pl.BlockSpec((G * CT, D), lambda e, mb: (e, 0)),
            pl.BlockSpec((1, D, MBK), lambda e, mb: (e, 0, mb)),
            pl.BlockSpec((1, 1, MBK), lambda e, mb: (e, 0, mb)),
            pl.BlockSpec((1, MBK, D), lambda e, mb: (e, mb, 0)),
            pl.BlockSpec((1, 1, D), lambda e, mb: (e, 0, 0)),
        ],
        out_specs=pl.BlockSpec((G * CT, D), lambda e, mb: (e, 0)),
        out_shape=jax.ShapeDtypeStruct((NVALID, D), jnp.float32),
        compiler_params=pltpu.CompilerParams(
            dimension_semantics=("parallel", "arbitrary")),
    )(xe, w1, b1.reshape(NE, 1, MLPD), w2, b2.reshape(NE, 1, D))


def _combine_body(ye_hbm, g0d_hbm, g1d_hbm, w0d_hbm, w1d_hbm, g0c_hbm,
                  g1c_hbm, w0c_hbm, w1c_hbm, outd_hbm, outc_hbm,
                  r0v, r1v, ov, iv0, iv1, wv0, wv1, gsem, wsem):
    c = lax.axis_index("c")
    s = lax.axis_index("s")
    w = s * NC + c

    def weighted_rows(n, r0, r1, roff, o, ooff, vw0, vw1, woff):
        def body(t, _):
            wb0 = vw0[woff + t, pl.ds(0, 16)]
            wb1 = vw1[woff + t, pl.ds(0, 16)]
            for j in range(D // 16):
                a = r0[roff + t, pl.ds(j * 16, 16)]
                b = r1[roff + t, pl.ds(j * 16, 16)]
                o[ooff + t, pl.ds(j * 16, 16)] = a * wb0 + b * wb1
            return 0
        lax.fori_loop(0, n, body, 0)

    toff = w * 128
    pltpu.sync_copy(g0d_hbm.at[pl.ds(toff, 128)], iv0)
    pltpu.sync_copy(g1d_hbm.at[pl.ds(toff, 128)], iv1)
    pltpu.sync_copy(w0d_hbm.at[pl.ds(toff, 128)], wv0)
    pltpu.sync_copy(w1d_hbm.at[pl.ds(toff, 128)], wv1)

    CH = 16
    def gath(ch, slot):
        return (pltpu.make_async_copy(
                    ye_hbm.at[iv0.at[pl.ds(ch * CH, CH)]],
                    r0v.at[pl.ds(slot * CH, CH)], gsem),
                pltpu.make_async_copy(
                    ye_hbm.at[iv1.at[pl.ds(ch * CH, CH)]],
                    r1v.at[pl.ds(slot * CH, CH)], gsem))

    def wrt(ch, slot):
        return pltpu.make_async_copy(
            ov.at[pl.ds(slot * CH, CH)],
            outd_hbm.at[pl.ds(toff + ch * CH, CH)], wsem)

    for c0, c1 in (gath(0, 0), gath(1, 1)):
        c0.start(); c1.start()
    for ch in range(8):
        gs = ch % 2
        os = ch % 2
        c0, c1 = gath(ch, gs)
        c0.wait(); c1.wait()
        if ch >= 2:
            wrt(ch - 2, os).wait()
        weighted_rows(CH, r0v, r1v, gs * CH, ov, os * CH, wv0, wv1, ch * CH)
        wrt(ch, os).start()
        if ch < 6:
            n0, n1 = gath(ch + 2, gs)
            n0.start(); n1.start()

    toff = w * 16
    pltpu.sync_copy(g0c_hbm.at[pl.ds(toff, 16)],
                    iv0.at[pl.ds(0, 16)])
    pltpu.sync_copy(g1c_hbm.at[pl.ds(toff, 16)], iv1.at[pl.ds(0, 16)])
    pltpu.sync_copy(w0c_hbm.at[pl.ds(toff, 16)], wv0.at[pl.ds(0, 16)])
    pltpu.sync_copy(w1c_hbm.at[pl.ds(toff, 16)], wv1.at[pl.ds(0, 16)])
    pltpu.sync_copy(ye_hbm.at[iv0.at[pl.ds(0, 16)]], r0v.at[pl.ds(0, 16)])
    pltpu.sync_copy(ye_hbm.at[iv1.at[pl.ds(0, 16)]], r1v.at[pl.ds(0, 16)])
    wrt(6, 0).wait()
    wrt(7, 1).wait()
    weighted_rows(16, r0v, r1v, 0, r0v, 16, wv0, wv1, 0)
    pltpu.sync_copy(r0v.at[pl.ds(16, 16)], outc_hbm.at[pl.ds(toff, 16)])


def _combine_call(ye, g0d, g1d, w0d, w1d, g0c, g1c, w0c, w1c):
    mesh = plsc.VectorSubcoreMesh(core_axis_name="c", subcore_axis_name="s")
    return pl.kernel(
        _combine_body,
        out_type=(jax.ShapeDtypeStruct((G * S_DET, D), jnp.float32),
                  jax.ShapeDtypeStruct((G * S_CLS, D), jnp.float32)),
        mesh=mesh,
        scratch_types=[
            pltpu.VMEM((32, D), jnp.float32),
            pltpu.VMEM((32, D), jnp.float32),
            pltpu.VMEM((32, D), jnp.float32),
            pltpu.VMEM((128,), jnp.int32),
            pltpu.VMEM((128,), jnp.int32),
            pltpu.VMEM((128, 16), jnp.float32),
            pltpu.VMEM((128, 16), jnp.float32),
            pltpu.SemaphoreType.DMA,
            pltpu.SemaphoreType.DMA,
        ],
    )(ye, g0d, g1d, w0d, w1d, g0c, g1c, w0c, w1c)


def kernel(inputs_det, inputs_cls, w_router_det, w_router_cls, w1, b1, w2, b2):
    xd2 = inputs_det.reshape(G * S_DET, D)
    xc2 = inputs_cls.reshape(G * S_CLS, D)
    d0d, d1d, g0d, g1d, w0d, w1d = _router_call(xd2, w_router_det,
                                                S_DET, C_DET, 0)
    d0c, d1c, g0c, g1c, w0c, w1c = _router_call(xc2, w_router_cls,
                                                S_CLS, C_CLS, C_DET)
    xe = _dispatch_call(xd2, xc2, d0d, d1d, d0c, d1c)
    ye = _mlp_call(xe, w1, b1, w2, b2)
    outd2, outc2 = _combine_call(ye, g0d, g1d, w0d, w1d, g0c, g1c, w0c, w1c)
    return outd2.reshape(inputs_det.shape), outc2.reshape(inputs_cls.shape)
